# baseline (speedup 1.0000x reference)
import jax
import jax.numpy as jnp
from jax import lax
from jax.experimental import pallas as pl
from jax.experimental.pallas import tpu as pltpu
from jax.experimental.pallas import tpu_sc as plsc

NUM_CORES = 2
NUM_SUBCORES = 16
LANES = 16
NUM_WORKERS = NUM_CORES * NUM_SUBCORES

BATCH = 16384
D = 32
B_PER_W = BATCH // NUM_WORKERS
N_GROUPS = B_PER_W // LANES


def _gmf_body(uidx_hbm, iidx_hbm, eu_hbm, ei_hbm, w_hbm, b_hbm, out_hbm,
              uidx_v, iidx_v, urows_v, irows_v, w_v, b_v, out_v, sem0, sem1):
    wid = lax.axis_index("s") * NUM_CORES + lax.axis_index("c")
    base = wid * B_PER_W

    pltpu.sync_copy(uidx_hbm.at[pl.ds(base, B_PER_W)], uidx_v)
    pltpu.sync_copy(iidx_hbm.at[pl.ds(base, B_PER_W)], iidx_v)
    pltpu.sync_copy(w_hbm, w_v)
    pltpu.sync_copy(b_hbm, b_v.at[pl.ds(0, 1)])

    w_lo = w_v[0, pl.ds(0, LANES)]
    w_hi = w_v[0, pl.ds(LANES, LANES)]
    bias_bc = jnp.broadcast_to(b_v[pl.ds(0, LANES)][0], (LANES,))
    lane = lax.iota(jnp.int32, LANES)

    def fire(g, sem, slot_base):
        uvec = uidx_v[pl.ds(pl.multiple_of(g * LANES, LANES), LANES)]
        ivec = iidx_v[pl.ds(pl.multiple_of(g * LANES, LANES), LANES)]
        for j in range(LANES):
            pltpu.async_copy(eu_hbm.at[pl.ds(uvec[j], 1)],
                             urows_v.at[pl.ds(slot_base + j, 1)], sem)
            pltpu.async_copy(ei_hbm.at[pl.ds(ivec[j], 1)],
                             irows_v.at[pl.ds(slot_base + j, 1)], sem)

    def drain_compute(g, sem, slot_base):
        pltpu.make_async_copy(eu_hbm.at[pl.ds(0, LANES)],
                              urows_v.at[pl.ds(slot_base, LANES)], sem).wait()
        pltpu.make_async_copy(ei_hbm.at[pl.ds(0, LANES)],
                              irows_v.at[pl.ds(slot_base, LANES)], sem).wait()
        acc = bias_bc
        for j in range(LANES):
            u0 = urows_v[slot_base + j, pl.ds(0, LANES)]
            u1 = urows_v[slot_base + j, pl.ds(LANES, LANES)]
            v0 = irows_v[slot_base + j, pl.ds(0, LANES)]
            v1 = irows_v[slot_base + j, pl.ds(LANES, LANES)]
            s = u0 * v0 * w_lo + u1 * v1 * w_hi
            acc = jnp.where(lane == j, bias_bc + jnp.sum(s), acc)
        out_v[pl.ds(pl.multiple_of(g * LANES, LANES), LANES)] = acc

    def it_body(t, _):
        t_even = (t % 2) == 0
        p_even = (t % 2) == 1

        @pl.when(jnp.logical_and(t < N_GROUPS, t_even))
        def _():
            fire(t, sem0, 0)

        @pl.when(jnp.logical_and(t < N_GROUPS, jnp.logical_not(t_even)))
        def _():
            fire(t, sem1, LANES)

        @pl.when(jnp.logical_and(t >= 1, p_even))
        def _():
            drain_compute(t - 1, sem0, 0)

        @pl.when(jnp.logical_and(t >= 1, jnp.logical_not(p_even)))
        def _():
            drain_compute(t - 1, sem1, LANES)

        return _

    lax.fori_loop(0, N_GROUPS + 1, it_body, None)

    pltpu.sync_copy(out_v, out_hbm.at[pl.ds(base, B_PER_W)])


def kernel(user_indices, item_indices, embed_user, embed_item, W_out, b_out):
    mesh = plsc.VectorSubcoreMesh(core_axis_name="c", subcore_axis_name="s",
                                  num_cores=NUM_CORES, num_subcores=NUM_SUBCORES)
    gmf = pl.kernel(
        _gmf_body,
        out_type=jax.ShapeDtypeStruct((BATCH,), jnp.float32),
        mesh=mesh,
        compiler_params=pltpu.CompilerParams(needs_layout_passes=False),
        scratch_types=[
            pltpu.VMEM((B_PER_W,), jnp.int32),
            pltpu.VMEM((B_PER_W,), jnp.int32),
            pltpu.VMEM((2 * LANES, D), jnp.float32),
            pltpu.VMEM((2 * LANES, D), jnp.float32),
            pltpu.VMEM((1, D), jnp.float32),
            pltpu.VMEM((LANES,), jnp.float32),
            pltpu.VMEM((B_PER_W,), jnp.float32),
            pltpu.SemaphoreType.DMA,
            pltpu.SemaphoreType.DMA,
        ],
    )
    return gmf(user_indices.astype(jnp.int32), item_indices.astype(jnp.int32),
               embed_user, embed_item, W_out, b_out)


# speedup vs baseline: 1.7927x; 1.7927x over previous
import jax
import jax.numpy as jnp
from jax import lax
from jax.experimental import pallas as pl
from jax.experimental.pallas import tpu as pltpu
from jax.experimental.pallas import tpu_sc as plsc

NUM_CORES = 2
NUM_SUBCORES = 16
LANES = 16
NUM_WORKERS = NUM_CORES * NUM_SUBCORES

BATCH = 16384
D = 32
TILE_W = 128
B_PER_W = BATCH // NUM_WORKERS
N_GROUPS = B_PER_W // LANES


def _gmf_body(uidx_hbm, iidx_hbm, eut_hbm, eit_hbm, w_hbm, b_hbm, out_hbm,
              uidx_v, iidx_v, utile_v, itile_v, w_v, b_v, out_v, sem0, sem1):
    wid = lax.axis_index("s") * NUM_CORES + lax.axis_index("c")
    base = wid * B_PER_W

    pltpu.sync_copy(uidx_hbm.at[pl.ds(base, B_PER_W)], uidx_v)
    pltpu.sync_copy(iidx_hbm.at[pl.ds(base, B_PER_W)], iidx_v)
    pltpu.sync_copy(w_hbm, w_v)
    pltpu.sync_copy(b_hbm, b_v.at[pl.ds(0, 1)])

    w_lo = w_v[0, pl.ds(0, LANES)]
    w_hi = w_v[0, pl.ds(LANES, LANES)]
    bias_bc = jnp.broadcast_to(b_v[pl.ds(0, LANES)][0], (LANES,))
    lane = lax.iota(jnp.int32, LANES)
    row_lo = lax.iota(jnp.int32, LANES)
    row_hi = row_lo + LANES
    sems = (sem0, sem1)

    def fire(ru, ri, slot):
        qu = pl.multiple_of((ru >> 7) * TILE_W, TILE_W)
        qi = pl.multiple_of((ri >> 7) * TILE_W, TILE_W)
        pltpu.async_copy(eut_hbm.at[pl.ds(0, D), pl.ds(qu, TILE_W)],
                         utile_v.at[pl.ds(0, D), pl.ds(slot * TILE_W, TILE_W)],
                         sems[slot])
        pltpu.async_copy(eit_hbm.at[pl.ds(0, D), pl.ds(qi, TILE_W)],
                         itile_v.at[pl.ds(0, D), pl.ds(slot * TILE_W, TILE_W)],
                         sems[slot])

    def drain(slot):
        pltpu.make_async_copy(
            eut_hbm.at[pl.ds(0, D), pl.ds(0, TILE_W)],
            utile_v.at[pl.ds(0, D), pl.ds(slot * TILE_W, TILE_W)],
            sems[slot]).wait()
        pltpu.make_async_copy(
            eit_hbm.at[pl.ds(0, D), pl.ds(0, TILE_W)],
            itile_v.at[pl.ds(0, D), pl.ds(slot * TILE_W, TILE_W)],
            sems[slot]).wait()

    def group_body(g, _):
        uvec = uidx_v[pl.ds(pl.multiple_of(g * LANES, LANES), LANES)]
        ivec = iidx_v[pl.ds(pl.multiple_of(g * LANES, LANES), LANES)]
        fire(uvec[0], ivec[0], 0)
        acc = bias_bc
        for j in range(LANES):
            if j + 1 < LANES:
                fire(uvec[j + 1], ivec[j + 1], (j + 1) % 2)
            drain(j % 2)
            cu = jnp.full((LANES,), (j % 2) * TILE_W, jnp.int32) + (uvec[j] & 127)
            ci = jnp.full((LANES,), (j % 2) * TILE_W, jnp.int32) + (ivec[j] & 127)
            u0 = plsc.load_gather(utile_v, [row_lo, cu])
            u1 = plsc.load_gather(utile_v, [row_hi, cu])
            v0 = plsc.load_gather(itile_v, [row_lo, ci])
            v1 = plsc.load_gather(itile_v, [row_hi, ci])
            s = u0 * v0 * w_lo + u1 * v1 * w_hi
            acc = jnp.where(lane == j, bias_bc + jnp.sum(s), acc)
        out_v[pl.ds(pl.multiple_of(g * LANES, LANES), LANES)] = acc
        return _

    lax.fori_loop(0, N_GROUPS, group_body, None)

    pltpu.sync_copy(out_v, out_hbm.at[pl.ds(base, B_PER_W)])


def kernel(user_indices, item_indices, embed_user, embed_item, W_out, b_out):
    mesh = plsc.VectorSubcoreMesh(core_axis_name="c", subcore_axis_name="s",
                                  num_cores=NUM_CORES, num_subcores=NUM_SUBCORES)
    gmf = pl.kernel(
        _gmf_body,
        out_type=jax.ShapeDtypeStruct((BATCH,), jnp.float32),
        mesh=mesh,
        compiler_params=pltpu.CompilerParams(needs_layout_passes=False),
        scratch_types=[
            pltpu.VMEM((B_PER_W,), jnp.int32),
            pltpu.VMEM((B_PER_W,), jnp.int32),
            pltpu.VMEM((D, 2 * TILE_W), jnp.float32),
            pltpu.VMEM((D, 2 * TILE_W), jnp.float32),
            pltpu.VMEM((1, D), jnp.float32),
            pltpu.VMEM((LANES,), jnp.float32),
            pltpu.VMEM((B_PER_W,), jnp.float32),
            pltpu.SemaphoreType.DMA,
            pltpu.SemaphoreType.DMA,
        ],
    )
    return gmf(user_indices.astype(jnp.int32), item_indices.astype(jnp.int32),
               embed_user.T, embed_item.T, W_out, b_out)


# speedup vs baseline: 2.2740x; 1.2685x over previous
import jax
import jax.numpy as jnp
from jax import lax
from jax.experimental import pallas as pl
from jax.experimental.pallas import tpu as pltpu
from jax.experimental.pallas import tpu_sc as plsc

NUM_CORES = 2
NUM_SUBCORES = 16
LANES = 16
NUM_WORKERS = NUM_CORES * NUM_SUBCORES

BATCH = 16384
D = 32
TILE_W = 128
B_PER_W = BATCH // NUM_WORKERS
N_GROUPS = B_PER_W // LANES


def _gmf_body(uidx_hbm, iidx_hbm, eut_hbm, eit_hbm, w_hbm, b_hbm, out_hbm,
              uidx_v, iidx_v, utile_v, itile_v, w_v, b_v, out_v,
              sem0, sem1, sem2, sem3):
    wid = lax.axis_index("s") * NUM_CORES + lax.axis_index("c")
    base = wid * B_PER_W

    pltpu.sync_copy(uidx_hbm.at[pl.ds(base, B_PER_W)], uidx_v)
    pltpu.sync_copy(iidx_hbm.at[pl.ds(base, B_PER_W)], iidx_v)
    pltpu.sync_copy(w_hbm, w_v)
    pltpu.sync_copy(b_hbm, b_v.at[pl.ds(0, 1)])

    w_lo = w_v[0, pl.ds(0, LANES)]
    w_hi = w_v[0, pl.ds(LANES, LANES)]
    bias_bc = jnp.broadcast_to(b_v[pl.ds(0, LANES)][0], (LANES,))
    lane = lax.iota(jnp.int32, LANES)
    row_lo = lax.iota(jnp.int32, LANES)
    row_hi = row_lo + LANES
    sems = (sem0, sem1, sem2, sem3)

    def fire(ru, ri, slot):
        qu = pl.multiple_of((ru >> 7) * TILE_W, TILE_W)
        qi = pl.multiple_of((ri >> 7) * TILE_W, TILE_W)
        pltpu.async_copy(eut_hbm.at[pl.ds(0, D), pl.ds(qu, TILE_W)],
                         utile_v.at[pl.ds(0, D), pl.ds(slot * TILE_W, TILE_W)],
                         sems[slot])
        pltpu.async_copy(eit_hbm.at[pl.ds(0, D), pl.ds(qi, TILE_W)],
                         itile_v.at[pl.ds(0, D), pl.ds(slot * TILE_W, TILE_W)],
                         sems[slot])

    def drain(slot):
        pltpu.make_async_copy(
            eut_hbm.at[pl.ds(0, D), pl.ds(0, TILE_W)],
            utile_v.at[pl.ds(0, D), pl.ds(slot * TILE_W, TILE_W)],
            sems[slot]).wait()
        pltpu.make_async_copy(
            eit_hbm.at[pl.ds(0, D), pl.ds(0, TILE_W)],
            itile_v.at[pl.ds(0, D), pl.ds(slot * TILE_W, TILE_W)],
            sems[slot]).wait()

    def group_body(g, _):
        uvec = uidx_v[pl.ds(pl.multiple_of(g * LANES, LANES), LANES)]
        ivec = iidx_v[pl.ds(pl.multiple_of(g * LANES, LANES), LANES)]
        for k in range(3):
            fire(uvec[k], ivec[k], k)
        acc = bias_bc
        for j in range(LANES):
            if j + 3 < LANES:
                fire(uvec[j + 3], ivec[j + 3], (j + 3) % 4)
            drain(j % 4)
            cu = jnp.full((LANES,), (j % 4) * TILE_W, jnp.int32) + (uvec[j] & 127)
            ci = jnp.full((LANES,), (j % 4) * TILE_W, jnp.int32) + (ivec[j] & 127)
            u0 = plsc.load_gather(utile_v, [row_lo, cu])
            u1 = plsc.load_gather(utile_v, [row_hi, cu])
            v0 = plsc.load_gather(itile_v, [row_lo, ci])
            v1 = plsc.load_gather(itile_v, [row_hi, ci])
            s = u0 * v0 * w_lo + u1 * v1 * w_hi
            acc = jnp.where(lane == j, bias_bc + jnp.sum(s), acc)
        out_v[pl.ds(pl.multiple_of(g * LANES, LANES), LANES)] = acc
        return _

    lax.fori_loop(0, N_GROUPS, group_body, None)

    pltpu.sync_copy(out_v, out_hbm.at[pl.ds(base, B_PER_W)])


def kernel(user_indices, item_indices, embed_user, embed_item, W_out, b_out):
    mesh = plsc.VectorSubcoreMesh(core_axis_name="c", subcore_axis_name="s",
                                  num_cores=NUM_CORES, num_subcores=NUM_SUBCORES)
    gmf = pl.kernel(
        _gmf_body,
        out_type=jax.ShapeDtypeStruct((BATCH,), jnp.float32),
        mesh=mesh,
        compiler_params=pltpu.CompilerParams(needs_layout_passes=False),
        scratch_types=[
            pltpu.VMEM((B_PER_W,), jnp.int32),
            pltpu.VMEM((B_PER_W,), jnp.int32),
            pltpu.VMEM((D, 4 * TILE_W), jnp.float32),
            pltpu.VMEM((D, 4 * TILE_W), jnp.float32),
            pltpu.VMEM((1, D), jnp.float32),
            pltpu.VMEM((LANES,), jnp.float32),
            pltpu.VMEM((B_PER_W,), jnp.float32),
            pltpu.SemaphoreType.DMA,
            pltpu.SemaphoreType.DMA,
            pltpu.SemaphoreType.DMA,
            pltpu.SemaphoreType.DMA,
        ],
    )
    return gmf(user_indices.astype(jnp.int32), item_indices.astype(jnp.int32),
               embed_user.T, embed_item.T, W_out, b_out)


# speedup vs baseline: 2.5458x; 1.1195x over previous
import jax
import jax.numpy as jnp
from jax import lax
from jax.experimental import pallas as pl
from jax.experimental.pallas import tpu as pltpu
from jax.experimental.pallas import tpu_sc as plsc

NUM_CORES = 2
NUM_SUBCORES = 16
LANES = 16
NUM_WORKERS = NUM_CORES * NUM_SUBCORES

BATCH = 16384
D = 32
TILE_W = 128
B_PER_W = BATCH // NUM_WORKERS
N_GROUPS = B_PER_W // LANES


def _gmf_body(uidx_hbm, iidx_hbm, eut_hbm, eit_hbm, w_hbm, b_hbm, out_hbm,
              uidx_v, iidx_v, utile_v, itile_v, w_v, b_v, out_v,
              sem0, sem1, sem2, sem3, sem4, sem5, sem6, sem7):
    wid = lax.axis_index("s") * NUM_CORES + lax.axis_index("c")
    base = wid * B_PER_W

    pltpu.sync_copy(uidx_hbm.at[pl.ds(base, B_PER_W)], uidx_v)
    pltpu.sync_copy(iidx_hbm.at[pl.ds(base, B_PER_W)], iidx_v)
    pltpu.sync_copy(w_hbm, w_v)
    pltpu.sync_copy(b_hbm, b_v.at[pl.ds(0, 1)])

    w_lo = w_v[0, pl.ds(0, LANES)]
    w_hi = w_v[0, pl.ds(LANES, LANES)]
    bias_bc = jnp.broadcast_to(b_v[pl.ds(0, LANES)][0], (LANES,))
    lane = lax.iota(jnp.int32, LANES)
    row_lo = lax.iota(jnp.int32, LANES)
    row_hi = row_lo + LANES
    sems = (sem0, sem1, sem2, sem3, sem4, sem5, sem6, sem7)

    def fire(ru, ri, slot):
        qu = pl.multiple_of((ru >> 7) * TILE_W, TILE_W)
        qi = pl.multiple_of((ri >> 7) * TILE_W, TILE_W)
        pltpu.async_copy(eut_hbm.at[pl.ds(0, D), pl.ds(qu, TILE_W)],
                         utile_v.at[pl.ds(0, D), pl.ds(slot * TILE_W, TILE_W)],
                         sems[slot])
        pltpu.async_copy(eit_hbm.at[pl.ds(0, D), pl.ds(qi, TILE_W)],
                         itile_v.at[pl.ds(0, D), pl.ds(slot * TILE_W, TILE_W)],
                         sems[slot])

    def drain(slot):
        pltpu.make_async_copy(
            eut_hbm.at[pl.ds(0, D), pl.ds(0, TILE_W)],
            utile_v.at[pl.ds(0, D), pl.ds(slot * TILE_W, TILE_W)],
            sems[slot]).wait()
        pltpu.make_async_copy(
            eit_hbm.at[pl.ds(0, D), pl.ds(0, TILE_W)],
            itile_v.at[pl.ds(0, D), pl.ds(slot * TILE_W, TILE_W)],
            sems[slot]).wait()

    def group_body(g, _):
        uvec = uidx_v[pl.ds(pl.multiple_of(g * LANES, LANES), LANES)]
        ivec = iidx_v[pl.ds(pl.multiple_of(g * LANES, LANES), LANES)]
        for k in range(7):
            fire(uvec[k], ivec[k], k)
        acc = bias_bc
        for j in range(LANES):
            if j + 7 < LANES:
                fire(uvec[j + 7], ivec[j + 7], (j + 7) % 8)
            drain(j % 8)
            cu = jnp.full((LANES,), (j % 8) * TILE_W, jnp.int32) + (uvec[j] & 127)
            ci = jnp.full((LANES,), (j % 8) * TILE_W, jnp.int32) + (ivec[j] & 127)
            u0 = plsc.load_gather(utile_v, [row_lo, cu])
            u1 = plsc.load_gather(utile_v, [row_hi, cu])
            v0 = plsc.load_gather(itile_v, [row_lo, ci])
            v1 = plsc.load_gather(itile_v, [row_hi, ci])
            s = u0 * v0 * w_lo + u1 * v1 * w_hi
            acc = jnp.where(lane == j, bias_bc + jnp.sum(s), acc)
        out_v[pl.ds(pl.multiple_of(g * LANES, LANES), LANES)] = acc
        return _

    lax.fori_loop(0, N_GROUPS, group_body, None)

    pltpu.sync_copy(out_v, out_hbm.at[pl.ds(base, B_PER_W)])


def kernel(user_indices, item_indices, embed_user, embed_item, W_out, b_out):
    mesh = plsc.VectorSubcoreMesh(core_axis_name="c", subcore_axis_name="s",
                                  num_cores=NUM_CORES, num_subcores=NUM_SUBCORES)
    gmf = pl.kernel(
        _gmf_body,
        out_type=jax.ShapeDtypeStruct((BATCH,), jnp.float32),
        mesh=mesh,
        compiler_params=pltpu.CompilerParams(needs_layout_passes=False),
        scratch_types=[
            pltpu.VMEM((B_PER_W,), jnp.int32),
            pltpu.VMEM((B_PER_W,), jnp.int32),
            pltpu.VMEM((D, 8 * TILE_W), jnp.float32),
            pltpu.VMEM((D, 8 * TILE_W), jnp.float32),
            pltpu.VMEM((1, D), jnp.float32),
            pltpu.VMEM((LANES,), jnp.float32),
            pltpu.VMEM((B_PER_W,), jnp.float32),
        ] + [pltpu.SemaphoreType.DMA] * 8,
    )
    return gmf(user_indices.astype(jnp.int32), item_indices.astype(jnp.int32),
               embed_user.T, embed_item.T, W_out, b_out)
